# baseline (speedup 1.0000x reference)
import functools

import jax
import jax.numpy as jnp
from jax import lax
from jax.experimental import pallas as pl
from jax.experimental.pallas import tpu as pltpu
from jax.experimental.pallas import tpu_sc as plsc

_TOKS = [345, 1101, 4523, 9872, 15000]
_B, _S, _V = 8, 512, 32000
_NT = len(_TOKS)
_L = 16
_W = 128
_ROWS = _B * _S * _V // _W
_SVW = _S * _V // _W
_VW = _V // _W
_NIDX = 48

_mesh = plsc.VectorSubcoreMesh(core_axis_name="c", subcore_axis_name="s")


def _sel5(jvec, table):
    x = jnp.where(jvec == 3, table[3], table[4])
    x = jnp.where(jvec == 2, table[2], x)
    x = jnp.where(jvec == 1, table[1], x)
    return jnp.where(jvec == 0, table[0], x)


@functools.partial(
    pl.kernel,
    mesh=_mesh,
    compiler_params=pltpu.CompilerParams(needs_layout_passes=False),
    out_type=jax.ShapeDtypeStruct((_NT * _L,), jnp.float32),
    scratch_types=[
        pltpu.VMEM((_S,), jnp.int32),
        pltpu.VMEM((2 * _L,), jnp.int32),
        pltpu.VMEM((_NIDX,), jnp.int32),
        pltpu.VMEM((_NIDX, _W), jnp.float32),
        pltpu.VMEM((_NT * _L,), jnp.float32),
        pltpu.SemaphoreType.DMA,
    ],
)
def _sc_probs(logits_hbm, mask_hbm, out_hbm, mrow_v, tree_v, idx_v, rows_v,
              res_v, sem):
    cid = lax.axis_index("c")
    sid = lax.axis_index("s")

    @pl.when(jnp.logical_and(cid == 0, sid == 0))
    def _():
        lane = lax.iota(jnp.int32, _L)
        zidx = lane * 0

        pltpu.sync_copy(mask_hbm.at[pl.ds(0, _S)], mrow_v)
        acc = mrow_v[pl.ds(0, _L)]
        for i in range(1, _S // _L):
            acc = acc + mrow_v[pl.ds(i * _L, _L)]
        for sh in (1, 2, 4, 8):
            tree_v[pl.ds(0, _L)] = acc
            tree_v[pl.ds(_L, _L)] = acc
            acc = acc + tree_v[pl.ds(sh, _L)]
        last_idx = acc - 1

        tokdiv = [t // _W for t in _TOKS]
        for c in range(_NIDX // _L):
            k = lane + c * _L
            bk = k // _NT
            jk = k - bk * _NT
            base = jnp.where(k < _B * _NT, bk * _SVW + _sel5(jk, tokdiv), 0)
            idx_v[pl.ds(c * _L, _L)] = base + last_idx * _VW

        pltpu.async_copy(logits_hbm.at[idx_v], rows_v, sem).wait()

        vals = []
        for j in range(_NT):
            rids = jnp.where(lane < _B, lane * _NT + j, 0)
            cidx = zidx + (_TOKS[j] % _W)
            vals.append(plsc.load_gather(rows_v, [rids, cidx]))
        m = vals[0]
        for j in range(1, _NT):
            m = jnp.maximum(m, vals[j])
        exps = [jnp.exp(v - m) for v in vals]
        s = exps[0]
        for j in range(1, _NT):
            s = s + exps[j]
        for j in range(_NT):
            res_v[pl.ds(j * _L, _L)] = exps[j] / s
        pltpu.sync_copy(res_v, out_hbm)


def kernel(logits, padding_mask):
    logits2d = logits.reshape(_ROWS, _W)
    out = _sc_probs(logits2d, padding_mask.reshape(-1))
    return out.reshape(_NT, _L)[:, :_B].T


# speedup vs baseline: 16.2072x; 16.2072x over previous
import functools

import jax
import jax.numpy as jnp
from jax import lax
from jax.experimental import pallas as pl
from jax.experimental.pallas import tpu as pltpu
from jax.experimental.pallas import tpu_sc as plsc

_TOKS = [345, 1101, 4523, 9872, 15000]
_B, _S, _V = 8, 512, 32000
_NT = len(_TOKS)
_L = 16
_W = 128

_mesh = plsc.VectorSubcoreMesh(core_axis_name="c", subcore_axis_name="s")


@functools.partial(
    pl.kernel,
    mesh=_mesh,
    compiler_params=pltpu.CompilerParams(needs_layout_passes=False),
    out_type=jax.ShapeDtypeStruct((_NT * _L,), jnp.float32),
    scratch_types=[
        pltpu.VMEM((_S,), jnp.int32),
        pltpu.VMEM((2 * _L,), jnp.int32),
        pltpu.VMEM((_L,), jnp.int32),
        pltpu.VMEM((_NT * _L, _W), jnp.float32),
        pltpu.VMEM((_NT * _L,), jnp.float32),
        pltpu.SemaphoreType.DMA,
    ],
)
def _sc_probs(logits_hbm, mask_hbm, out_hbm, mrow_v, tree_v, idx_v, rows_v,
              res_v, sem):
    cid = lax.axis_index("c")
    sid = lax.axis_index("s")

    @pl.when(jnp.logical_and(cid == 0, sid == 0))
    def _():
        lane = lax.iota(jnp.int32, _L)

        pltpu.sync_copy(mask_hbm.at[pl.ds(0, _S)], mrow_v)
        acc = mrow_v[pl.ds(0, _L)]
        for i in range(1, _S // _L):
            acc = acc + mrow_v[pl.ds(i * _L, _L)]
        for sh in (1, 2, 4, 8):
            tree_v[pl.ds(0, _L)] = acc
            tree_v[pl.ds(_L, _L)] = acc
            acc = acc + tree_v[pl.ds(sh, _L)]
        last_idx = acc - 1

        bvec = jnp.where(lane < _B, lane, 0)
        idx_v[pl.ds(0, _L)] = bvec * _S + last_idx

        copies = []
        for j in range(_NT):
            win = logits_hbm.at[:, pl.ds((_TOKS[j] // _W) * _W, _W)]
            copies.append(
                pltpu.make_async_copy(
                    win.at[idx_v], rows_v.at[pl.ds(j * _L, _L)], sem))
        for c in copies:
            c.start()
        for c in copies:
            c.wait()

        vals = []
        for j in range(_NT):
            rids = j * _L + bvec
            cidx = lane * 0 + (_TOKS[j] % _W)
            vals.append(plsc.load_gather(rows_v, [rids, cidx]))
        m = vals[0]
        for j in range(1, _NT):
            m = jnp.maximum(m, vals[j])
        exps = [jnp.exp(v - m) for v in vals]
        s = exps[0]
        for j in range(1, _NT):
            s = s + exps[j]
        for j in range(_NT):
            res_v[pl.ds(j * _L, _L)] = exps[j] / s
        pltpu.sync_copy(res_v, out_hbm)


def kernel(logits, padding_mask):
    logits2d = logits.reshape(_B * _S, _V)
    out = _sc_probs(logits2d, padding_mask.reshape(-1))
    return out.reshape(_NT, _L)[:, :_B].T


# speedup vs baseline: 17.3900x; 1.0730x over previous
import functools

import jax
import jax.numpy as jnp
from jax import lax
from jax.experimental import pallas as pl
from jax.experimental.pallas import tpu as pltpu
from jax.experimental.pallas import tpu_sc as plsc

_TOKS = [345, 1101, 4523, 9872, 15000]
_B, _S, _V = 8, 512, 32000
_NT = len(_TOKS)
_L = 16
_W = 128

_mesh = plsc.VectorSubcoreMesh(
    core_axis_name="c", subcore_axis_name="s", num_cores=1)


@functools.partial(
    pl.kernel,
    mesh=_mesh,
    compiler_params=pltpu.CompilerParams(needs_layout_passes=False),
    out_type=jax.ShapeDtypeStruct((_NT * _L,), jnp.float32),
    scratch_types=[
        pltpu.VMEM((_S,), jnp.int32),
        pltpu.VMEM((2 * _L,), jnp.int32),
        pltpu.VMEM((_L,), jnp.int32),
        pltpu.VMEM((_NT * _L, _W), jnp.float32),
        pltpu.VMEM((_NT * _L,), jnp.float32),
        pltpu.SemaphoreType.DMA,
    ],
)
def _sc_probs(logits_hbm, mask_hbm, out_hbm, mrow_v, tree_v, idx_v, rows_v,
              res_v, sem):
    cid = lax.axis_index("c")
    sid = lax.axis_index("s")

    @pl.when(jnp.logical_and(cid == 0, sid == 0))
    def _():
        lane = lax.iota(jnp.int32, _L)

        pltpu.sync_copy(mask_hbm.at[pl.ds(0, _S)], mrow_v)
        acc = mrow_v[pl.ds(0, _L)]
        for i in range(1, _S // _L):
            acc = acc + mrow_v[pl.ds(i * _L, _L)]
        for sh in (1, 2, 4, 8):
            tree_v[pl.ds(0, _L)] = acc
            tree_v[pl.ds(_L, _L)] = acc
            acc = acc + tree_v[pl.ds(sh, _L)]
        last_idx = acc - 1

        bvec = jnp.where(lane < _B, lane, 0)
        idx_v[pl.ds(0, _L)] = bvec * _S + last_idx

        copies = []
        for j in range(_NT):
            win = logits_hbm.at[:, pl.ds((_TOKS[j] // _W) * _W, _W)]
            copies.append(
                pltpu.make_async_copy(
                    win.at[idx_v], rows_v.at[pl.ds(j * _L, _L)], sem))
        for c in copies:
            c.start()
        for c in copies:
            c.wait()

        vals = []
        for j in range(_NT):
            rids = j * _L + bvec
            cidx = lane * 0 + (_TOKS[j] % _W)
            vals.append(plsc.load_gather(rows_v, [rids, cidx]))
        m = vals[0]
        for j in range(1, _NT):
            m = jnp.maximum(m, vals[j])
        exps = [jnp.exp(v - m) for v in vals]
        s = exps[0]
        for j in range(1, _NT):
            s = s + exps[j]
        for j in range(_NT):
            res_v[pl.ds(j * _L, _L)] = exps[j] / s
        pltpu.sync_copy(res_v, out_hbm)


def kernel(logits, padding_mask):
    logits2d = logits.reshape(_B * _S, _V)
    out = _sc_probs(logits2d, padding_mask.reshape(-1))
    return out.reshape(_NT, _L)[:, :_B].T


# speedup vs baseline: 18.2211x; 1.0478x over previous
import functools

import jax
import jax.numpy as jnp
from jax import lax
from jax.experimental import pallas as pl
from jax.experimental.pallas import tpu as pltpu
from jax.experimental.pallas import tpu_sc as plsc

_TOKS = [345, 1101, 4523, 9872, 15000]
_B, _S, _V = 8, 512, 32000
_NT = len(_TOKS)
_L = 16
_W = 128

_mesh = plsc.VectorSubcoreMesh(
    core_axis_name="c", subcore_axis_name="s", num_cores=1)


@functools.partial(
    pl.kernel,
    mesh=_mesh,
    compiler_params=pltpu.CompilerParams(needs_layout_passes=False),
    out_type=jax.ShapeDtypeStruct((_NT * _L,), jnp.float32),
    scratch_types=[
        pltpu.VMEM((_S,), jnp.int32),
        pltpu.VMEM((_NT * _B, _W), jnp.float32),
        pltpu.VMEM((_NT * _L,), jnp.float32),
        pltpu.SemaphoreType.DMA,
    ],
)
def _sc_probs(logits_hbm, mask_hbm, out_hbm, mrow_v, rows_v, res_v, sem):
    cid = lax.axis_index("c")
    sid = lax.axis_index("s")

    @pl.when(jnp.logical_and(cid == 0, sid == 0))
    def _():
        lane = lax.iota(jnp.int32, _L)

        pltpu.sync_copy(mask_hbm.at[pl.ds(0, _S)], mrow_v)
        acc = mrow_v[pl.ds(0, _L)]
        for i in range(1, _S // _L):
            acc = acc + mrow_v[pl.ds(i * _L, _L)]
        total = acc[0]
        for i in range(1, _L):
            total = total + acc[i]
        last_idx = total - 1

        copies = []
        for j in range(_NT):
            src = logits_hbm.at[:, last_idx, pl.ds((_TOKS[j] // _W) * _W, _W)]
            copies.append(
                pltpu.make_async_copy(src, rows_v.at[pl.ds(j * _B, _B)], sem))
        for c in copies:
            c.start()
        for c in copies:
            c.wait()

        bvec = jnp.where(lane < _B, lane, 0)
        vals = []
        for j in range(_NT):
            rids = j * _B + bvec
            cidx = lane * 0 + (_TOKS[j] % _W)
            vals.append(plsc.load_gather(rows_v, [rids, cidx]))
        m = vals[0]
        for j in range(1, _NT):
            m = jnp.maximum(m, vals[j])
        exps = [jnp.exp(v - m) for v in vals]
        s = exps[0]
        for j in range(1, _NT):
            s = s + exps[j]
        for j in range(_NT):
            res_v[pl.ds(j * _L, _L)] = exps[j] / s
        pltpu.sync_copy(res_v, out_hbm)


def kernel(logits, padding_mask):
    out = _sc_probs(logits, padding_mask.reshape(-1))
    return out.reshape(_NT, _L)[:, :_B].T


# speedup vs baseline: 19.3046x; 1.0595x over previous
import functools

import jax
import jax.numpy as jnp
from jax import lax
from jax.experimental import pallas as pl
from jax.experimental.pallas import tpu as pltpu
from jax.experimental.pallas import tpu_sc as plsc

_TOKS = [345, 1101, 4523, 9872, 15000]
_B, _S, _V = 8, 512, 32000
_NT = len(_TOKS)
_L = 16
_W = 128

_mesh = plsc.VectorSubcoreMesh(
    core_axis_name="c", subcore_axis_name="s", num_cores=1)


@functools.partial(
    pl.kernel,
    mesh=_mesh,
    compiler_params=pltpu.CompilerParams(needs_layout_passes=False),
    out_type=jax.ShapeDtypeStruct((_B, _NT), jnp.float32),
    scratch_types=[
        pltpu.VMEM((_S,), jnp.int32),
        pltpu.VMEM((_NT * _B, _W), jnp.float32),
        pltpu.VMEM((_B, _NT), jnp.float32),
        pltpu.SemaphoreType.DMA,
    ],
)
def _sc_probs(logits_hbm, mask_hbm, out_hbm, mrow_v, rows_v, res_v, sem):
    cid = lax.axis_index("c")
    sid = lax.axis_index("s")

    @pl.when(jnp.logical_and(cid == 0, sid == 0))
    def _():
        lane = lax.iota(jnp.int32, _L)

        pltpu.sync_copy(mask_hbm.at[0, pl.ds(0, _S)], mrow_v)
        acc = mrow_v[pl.ds(0, _L)]
        for i in range(1, _S // _L):
            acc = acc + mrow_v[pl.ds(i * _L, _L)]
        total = acc[0]
        for i in range(1, _L):
            total = total + acc[i]
        last_idx = total - 1

        copies = []
        for j in range(_NT):
            src = logits_hbm.at[:, last_idx, pl.ds((_TOKS[j] // _W) * _W, _W)]
            copies.append(
                pltpu.make_async_copy(src, rows_v.at[pl.ds(j * _B, _B)], sem))
        for c in copies:
            c.start()
        for c in copies:
            c.wait()

        bvec = jnp.where(lane < _B, lane, 0)
        vals = []
        for j in range(_NT):
            rids = j * _B + bvec
            cidx = lane * 0 + (_TOKS[j] % _W)
            vals.append(plsc.load_gather(rows_v, [rids, cidx]))
        m = vals[0]
        for j in range(1, _NT):
            m = jnp.maximum(m, vals[j])
        exps = [jnp.exp(v - m) for v in vals]
        s = exps[0]
        for j in range(1, _NT):
            s = s + exps[j]
        valid = lane < _B
        for j in range(_NT):
            plsc.store_scatter(res_v, [bvec, lane * 0 + j], exps[j] / s,
                               mask=valid)
        pltpu.sync_copy(res_v, out_hbm)


def kernel(logits, padding_mask):
    return _sc_probs(logits, padding_mask)


# speedup vs baseline: 19.9398x; 1.0329x over previous
import functools

import jax
import jax.numpy as jnp
from jax import lax
from jax.experimental import pallas as pl
from jax.experimental.pallas import tpu as pltpu
from jax.experimental.pallas import tpu_sc as plsc

_TOKS = [345, 1101, 4523, 9872, 15000]
_B, _S, _V = 8, 512, 32000
_NT = len(_TOKS)
_L = 16
_W = 128

_mesh = plsc.VectorSubcoreMesh(
    core_axis_name="c", subcore_axis_name="s", num_cores=1)


@functools.partial(
    pl.kernel,
    mesh=_mesh,
    compiler_params=pltpu.CompilerParams(needs_layout_passes=False),
    out_type=jax.ShapeDtypeStruct((_B, _NT), jnp.float32),
    scratch_types=[
        pltpu.VMEM((_S,), jnp.int32),
        pltpu.VMEM((_NT * _B, _W), jnp.float32),
        pltpu.VMEM((_B, _NT), jnp.float32),
        pltpu.SemaphoreType.DMA,
    ],
)
def _sc_probs(logits_hbm, mask_hbm, out_hbm, mrow_v, rows_v, res_v, sem):
    cid = lax.axis_index("c")
    sid = lax.axis_index("s")

    @pl.when(jnp.logical_and(cid == 0, sid == 0))
    def _():
        lane = lax.iota(jnp.int32, _L)

        def start_gathers(idx):
            copies = []
            for j in range(_NT):
                src = logits_hbm.at[:, idx, pl.ds((_TOKS[j] // _W) * _W, _W)]
                copies.append(pltpu.make_async_copy(
                    src, rows_v.at[pl.ds(j * _B, _B)], sem))
            for c in copies:
                c.start()
            return copies

        spec = start_gathers(_S - 1)

        pltpu.sync_copy(mask_hbm.at[0, pl.ds(0, _S)], mrow_v)
        acc = mrow_v[pl.ds(0, _L)]
        for i in range(1, _S // _L):
            acc = acc + mrow_v[pl.ds(i * _L, _L)]
        total = acc[0]
        for i in range(1, _L):
            total = total + acc[i]
        last_idx = total - 1
        for c in spec:
            c.wait()

        @pl.when(last_idx != _S - 1)
        def _regather():
            for c in start_gathers(last_idx):
                c.wait()

        bvec = jnp.where(lane < _B, lane, 0)
        vals = []
        for j in range(_NT):
            rids = j * _B + bvec
            cidx = lane * 0 + (_TOKS[j] % _W)
            vals.append(plsc.load_gather(rows_v, [rids, cidx]))
        m = vals[0]
        for j in range(1, _NT):
            m = jnp.maximum(m, vals[j])
        exps = [jnp.exp(v - m) for v in vals]
        s = exps[0]
        for j in range(1, _NT):
            s = s + exps[j]
        valid = lane < _B
        for j in range(_NT):
            plsc.store_scatter(res_v, [bvec, lane * 0 + j], exps[j] / s,
                               mask=valid)
        pltpu.sync_copy(res_v, out_hbm)


def kernel(logits, padding_mask):
    return _sc_probs(logits, padding_mask)
